# baseline (speedup 1.0000x reference)
import jax
import jax.numpy as jnp
from jax.experimental import pallas as pl
from jax.experimental.pallas import tpu as pltpu


def _moe_kernel(x_ref, rw_ref, w1_hbm, wg_hbm, w2_hbm, out_ref,
                w1b, wgb, w2b, cwv, schedv, nv, sched_smem, n_smem,
                sems, ssem):
    x = x_ref[...]
    rw = rw_ref[...]
    logits = jax.lax.dot_general(
        rw, x, (((1,), (1,)), ((), ())), preferred_element_type=jnp.float32)
    e, t = logits.shape
    row = jax.lax.broadcasted_iota(jnp.int32, (e, t), 0)
    m1 = jnp.max(logits, axis=0, keepdims=True)
    i1 = jnp.min(jnp.where(logits == m1, row, e), axis=0, keepdims=True)
    masked = jnp.where(row == i1, -jnp.inf, logits)
    m2 = jnp.max(masked, axis=0, keepdims=True)
    i2 = jnp.min(jnp.where(masked == m2, row, e), axis=0, keepdims=True)
    b = jnp.exp(m2 - m1)
    w_hi = 1.0 / (1.0 + b)
    w_lo = b / (1.0 + b)
    comb = (jnp.where(row == i1, w_hi, 0.0)
            + jnp.where(row == i2, w_lo, 0.0))

    sel = jnp.max(comb, axis=1, keepdims=True) > 0.0
    ee_r = jax.lax.broadcasted_iota(jnp.int32, (e, e), 0)
    ee_c = jax.lax.broadcasted_iota(jnp.int32, (e, e), 1)
    tri = (ee_r >= ee_c).astype(jnp.float32)
    pos = jax.lax.dot_general(
        tri, sel.astype(jnp.float32), (((1,), (0,)), ((), ())),
        preferred_element_type=jnp.float32).astype(jnp.int32)
    hits = jnp.logical_and(sel, (pos - 1) == ee_c)
    sched = jnp.sum(jnp.where(hits, ee_r, 0), axis=0, keepdims=True)
    nv[...] = pos[e - 1:e, :]
    schedv[...] = sched
    cwv[...] = jax.lax.dot_general(
        hits.astype(jnp.float32), comb, (((0,), (0,)), ((), ())),
        preferred_element_type=jnp.float32)

    cps = pltpu.make_async_copy(schedv, sched_smem, ssem.at[0])
    cpn = pltpu.make_async_copy(nv, n_smem, ssem.at[1])
    cps.start()
    cpn.start()
    cps.wait()
    cpn.wait()
    n = n_smem[0, 0]

    def start_slot(j, slot):
        ej = sched_smem[0, j]
        pltpu.make_async_copy(w1_hbm.at[ej], w1b.at[slot], sems.at[slot, 0]).start()
        pltpu.make_async_copy(wg_hbm.at[ej], wgb.at[slot], sems.at[slot, 1]).start()
        pltpu.make_async_copy(w2_hbm.at[ej], w2b.at[slot], sems.at[slot, 2]).start()

    def wait_slot(j, slot):
        ej = sched_smem[0, j]
        pltpu.make_async_copy(w1_hbm.at[ej], w1b.at[slot], sems.at[slot, 0]).wait()
        pltpu.make_async_copy(wg_hbm.at[ej], wgb.at[slot], sems.at[slot, 1]).wait()
        pltpu.make_async_copy(w2_hbm.at[ej], w2b.at[slot], sems.at[slot, 2]).wait()

    start_slot(0, 0)

    @pl.when(n > 1)
    def _():
        start_slot(1, 1)

    @pl.when(n > 2)
    def _():
        start_slot(2, 2)

    out_ref[...] = jnp.zeros_like(out_ref)

    def body(j, carry):
        slot = jax.lax.rem(j, 4)

        @pl.when(j + 3 < n)
        def _():
            start_slot(j + 3, jax.lax.rem(j + 3, 4))

        wait_slot(j, slot)
        h1 = jnp.dot(x, w1b[slot], preferred_element_type=jnp.float32)
        g = jnp.dot(x, wgb[slot], preferred_element_type=jnp.float32)
        act = h1 * (g * jax.nn.sigmoid(g))
        oe = jnp.dot(act, w2b[slot], preferred_element_type=jnp.float32)
        ccol = jnp.transpose(cwv[pl.ds(j, 1), :])
        out_ref[...] += ccol * oe
        return carry

    jax.lax.fori_loop(0, n, body, 0)


def kernel(x, router_w, w1, w_gate, w2):
    orig_shape = x.shape
    d = x.shape[-1]
    xf = x.reshape(-1, d)
    t = xf.shape[0]
    e = router_w.shape[0]
    h = w1.shape[2]

    out = pl.pallas_call(
        _moe_kernel,
        in_specs=[
            pl.BlockSpec(memory_space=pltpu.VMEM),
            pl.BlockSpec(memory_space=pltpu.VMEM),
            pl.BlockSpec(memory_space=pltpu.HBM),
            pl.BlockSpec(memory_space=pltpu.HBM),
            pl.BlockSpec(memory_space=pltpu.HBM),
        ],
        out_specs=pl.BlockSpec(memory_space=pltpu.VMEM),
        out_shape=jax.ShapeDtypeStruct((t, d), jnp.float32),
        scratch_shapes=[
            pltpu.VMEM((4, d, h), jnp.float32),
            pltpu.VMEM((4, d, h), jnp.float32),
            pltpu.VMEM((4, h, d), jnp.float32),
            pltpu.VMEM((e, t), jnp.float32),
            pltpu.VMEM((1, e), jnp.int32),
            pltpu.VMEM((1, 1), jnp.int32),
            pltpu.SMEM((1, e), jnp.int32),
            pltpu.SMEM((1, 1), jnp.int32),
            pltpu.SemaphoreType.DMA((4, 3)),
            pltpu.SemaphoreType.DMA((2,)),
        ],
    )(xf, router_w, w1, w_gate, w2)
    return out.reshape(orig_shape)


# speedup vs baseline: 1.0059x; 1.0059x over previous
import jax
import jax.numpy as jnp
from jax.experimental import pallas as pl
from jax.experimental.pallas import tpu as pltpu


def _moe_kernel(x_ref, rw_ref, w1_hbm, wg_hbm, w2_hbm, out_ref,
                w1b, wgb, w2b, cwv, schedv, nv, sched_smem, n_smem,
                sems, ssem):
    x = x_ref[...]
    rw = rw_ref[...]
    logits = jax.lax.dot_general(
        rw, x, (((1,), (1,)), ((), ())), preferred_element_type=jnp.float32)
    e, t = logits.shape
    row = jax.lax.broadcasted_iota(jnp.int32, (e, t), 0)
    m1 = jnp.max(logits, axis=0, keepdims=True)
    i1 = jnp.min(jnp.where(logits == m1, row, e), axis=0, keepdims=True)
    masked = jnp.where(row == i1, -jnp.inf, logits)
    m2 = jnp.max(masked, axis=0, keepdims=True)
    i2 = jnp.min(jnp.where(masked == m2, row, e), axis=0, keepdims=True)
    b = jnp.exp(m2 - m1)
    w_hi = 1.0 / (1.0 + b)
    w_lo = b / (1.0 + b)
    comb = (jnp.where(row == i1, w_hi, 0.0)
            + jnp.where(row == i2, w_lo, 0.0))

    sel = jnp.max(comb, axis=1, keepdims=True) > 0.0
    ee_r = jax.lax.broadcasted_iota(jnp.int32, (e, e), 0)
    ee_c = jax.lax.broadcasted_iota(jnp.int32, (e, e), 1)
    tri = (ee_r >= ee_c).astype(jnp.float32)
    pos = jax.lax.dot_general(
        tri, sel.astype(jnp.float32), (((1,), (0,)), ((), ())),
        preferred_element_type=jnp.float32).astype(jnp.int32)
    hits = jnp.logical_and(sel, (pos - 1) == ee_c)
    sched = jnp.sum(jnp.where(hits, ee_r, 0), axis=0, keepdims=True)
    nv[...] = pos[e - 1:e, :]
    schedv[...] = sched
    cps = pltpu.make_async_copy(schedv, sched_smem, ssem.at[0])
    cpn = pltpu.make_async_copy(nv, n_smem, ssem.at[1])
    cps.start()
    cpn.start()
    cwv[...] = jax.lax.dot_general(
        hits.astype(jnp.float32), comb, (((0,), (0,)), ((), ())),
        preferred_element_type=jnp.float32)
    cps.wait()
    cpn.wait()
    n = n_smem[0, 0]

    def start_slot(j, slot):
        ej = sched_smem[0, j]
        pltpu.make_async_copy(w1_hbm.at[ej], w1b.at[slot], sems.at[slot, 0]).start()
        pltpu.make_async_copy(wg_hbm.at[ej], wgb.at[slot], sems.at[slot, 1]).start()
        pltpu.make_async_copy(w2_hbm.at[ej], w2b.at[slot], sems.at[slot, 2]).start()

    def wait_slot(j, slot):
        ej = sched_smem[0, j]
        pltpu.make_async_copy(w1_hbm.at[ej], w1b.at[slot], sems.at[slot, 0]).wait()
        pltpu.make_async_copy(wg_hbm.at[ej], wgb.at[slot], sems.at[slot, 1]).wait()
        pltpu.make_async_copy(w2_hbm.at[ej], w2b.at[slot], sems.at[slot, 2]).wait()

    start_slot(0, 0)
    out_ref[...] = jnp.zeros_like(out_ref)

    def body(j, carry):
        slot = jax.lax.rem(j, 2)

        @pl.when(j + 1 < n)
        def _():
            start_slot(j + 1, 1 - slot)

        wait_slot(j, slot)
        h1 = jnp.dot(x, w1b[slot], preferred_element_type=jnp.float32)
        g = jnp.dot(x, wgb[slot], preferred_element_type=jnp.float32)
        act = h1 * (g * jax.nn.sigmoid(g))
        oe = jnp.dot(act, w2b[slot], preferred_element_type=jnp.float32)
        ccol = jnp.transpose(cwv[pl.ds(j, 1), :])
        out_ref[...] += ccol * oe
        return carry

    jax.lax.fori_loop(0, n, body, 0)


def kernel(x, router_w, w1, w_gate, w2):
    orig_shape = x.shape
    d = x.shape[-1]
    xf = x.reshape(-1, d)
    t = xf.shape[0]
    e = router_w.shape[0]
    h = w1.shape[2]

    out = pl.pallas_call(
        _moe_kernel,
        in_specs=[
            pl.BlockSpec(memory_space=pltpu.VMEM),
            pl.BlockSpec(memory_space=pltpu.VMEM),
            pl.BlockSpec(memory_space=pltpu.HBM),
            pl.BlockSpec(memory_space=pltpu.HBM),
            pl.BlockSpec(memory_space=pltpu.HBM),
        ],
        out_specs=pl.BlockSpec(memory_space=pltpu.VMEM),
        out_shape=jax.ShapeDtypeStruct((t, d), jnp.float32),
        scratch_shapes=[
            pltpu.VMEM((2, d, h), jnp.float32),
            pltpu.VMEM((2, d, h), jnp.float32),
            pltpu.VMEM((2, h, d), jnp.float32),
            pltpu.VMEM((e, t), jnp.float32),
            pltpu.VMEM((1, e), jnp.int32),
            pltpu.VMEM((1, 1), jnp.int32),
            pltpu.SMEM((1, e), jnp.int32),
            pltpu.SMEM((1, 1), jnp.int32),
            pltpu.SemaphoreType.DMA((2, 3)),
            pltpu.SemaphoreType.DMA((2,)),
        ],
    )(xf, router_w, w1, w_gate, w2)
    return out.reshape(orig_shape)
